# baseline (speedup 1.0000x reference)
import functools

import jax
import jax.numpy as jnp
from jax import lax
from jax.experimental import pallas as pl
from jax.experimental.pallas import tpu as pltpu
from jax.experimental.pallas import tpu_sc as plsc

H = 256
HP = H // 2
L = 16
NBB = H // 32
K = 128
CPW = 40
EPW = K * CPW

_GATHER_DNUMS = lax.GatherDimensionNumbers(
    offset_dims=(), collapsed_slice_dims=(0,), start_index_map=(0,)
)


def _lane_shuffle(x, perm):
    return lax.gather(
        x, perm[:, None], _GATHER_DNUMS, slice_sizes=(1,),
        mode=lax.GatherScatterMode.PROMISE_IN_BOUNDS,
    )


def _lane_sum(x, lane):
    for sh in (8, 4, 2, 1):
        x = x + _lane_shuffle(x, (lane + sh) & (L - 1))
    return x


def _bf16x2_to_f32(x_bf32):
    xi = plsc.bitcast(x_bf32, jnp.int32)
    even = plsc.bitcast(lax.shift_left(xi, 16), jnp.float32)
    odd = plsc.bitcast(
        lax.bitwise_and(xi, jnp.int32(-65536)), jnp.float32
    )
    return even, odd


def _pack_halves(x):
    lo = lax.bitcast_convert_type(
        x[:, :HP].astype(jnp.bfloat16), jnp.uint16
    ).astype(jnp.int32)
    hi = lax.bitcast_convert_type(
        x[:, HP:].astype(jnp.bfloat16), jnp.uint16
    ).astype(jnp.int32)
    return lo | (hi << 16)


def _tc_body(h_ref, wa_ref, wb_ref, b1_ref, u_ref, v_ref):
    hb = h_ref[...]
    u_ref[...] = _pack_halves(
        jnp.dot(hb, wa_ref[...], preferred_element_type=jnp.float32)
    )
    v_ref[...] = _pack_halves(
        jnp.dot(hb, wb_ref[...], preferred_element_type=jnp.float32)
        + b1_ref[...]
    )


def _node_tables(h, waT, wbT, b1):
    n = h.shape[0]
    blk = 1000
    grid = n // blk
    return pl.pallas_call(
        _tc_body,
        grid=(grid,),
        in_specs=[
            pl.BlockSpec((blk, H), lambda i: (i, 0)),
            pl.BlockSpec((H, H), lambda i: (0, 0)),
            pl.BlockSpec((H, H), lambda i: (0, 0)),
            pl.BlockSpec((1, H), lambda i: (0, 0)),
        ],
        out_specs=[
            pl.BlockSpec((blk, HP), lambda i: (i, 0)),
            pl.BlockSpec((blk, HP), lambda i: (i, 0)),
        ],
        out_shape=[
            jax.ShapeDtypeStruct((n, HP), jnp.int32),
            jax.ShapeDtypeStruct((n, HP), jnp.int32),
        ],
    )(h, waT, wbT, b1)


def _sc_edge_kernel(base_edge, n_out):
    info = plsc.get_sparse_core_info()
    nc, ns = info.num_cores, info.num_subcores
    nw = ns

    mesh = plsc.VectorSubcoreMesh(core_axis_name="c", subcore_axis_name="s")

    @functools.partial(
        pl.kernel,
        out_type=jax.ShapeDtypeStruct((n_out,), jnp.float32),
        mesh=mesh,
        compiler_params=pltpu.CompilerParams(needs_layout_passes=False),
        scratch_types=[
            pltpu.VMEM((2, EPW), jnp.int32),
            pltpu.VMEM((K, HP), jnp.int32),
            pltpu.VMEM((K, HP), jnp.int32),
            pltpu.VMEM((K, HP), jnp.int32),
            pltpu.VMEM((K, HP), jnp.int32),
            pltpu.VMEM((HP,), jnp.int32),
            pltpu.VMEM((L,), jnp.float32),
            pltpu.VMEM((EPW,), jnp.float32),
            pltpu.SemaphoreType.DMA,
            pltpu.SemaphoreType.DMA,
            pltpu.SemaphoreType.DMA,
            pltpu.SemaphoreType.DMA,
        ],
    )
    def sc_kernel(u_hbm, v_hbm, ei_hbm, w2_hbm, b2_hbm, out_hbm,
                  idxall, ua, ub, va, vb, w2v, b2v, outall,
                  su0, su1, sv0, sv1):
        cid = lax.axis_index("c")
        wid = lax.axis_index("s")
        span = base_edge + wid * EPW

        @pl.when(cid == 0)
        def _():
            pltpu.sync_copy(w2_hbm, w2v)
            pltpu.sync_copy(b2_hbm, b2v)
            pltpu.sync_copy(ei_hbm.at[:, pl.ds(span, EPW)], idxall)

        lane = lax.iota(jnp.int32, L)
        b2vec = b2v[...]
        w2a = []
        w2b = []
        for j in range(NBB):
            w2bf = plsc.bitcast(w2v[pl.ds(L * j, L)], jnp.bfloat16)
            a, b = plsc.unpack(
                w2bf,
                format=plsc.PackFormat.INTERLEAVED,
                preferred_element_type=jnp.float32,
            )
            w2a.append(a)
            w2b.append(b)

        def issue(c, ur, vr, su, sv):
            src = idxall.at[0, pl.ds(c * K, K)]
            dst = idxall.at[1, pl.ds(c * K, K)]
            pltpu.async_copy(u_hbm.at[src], ur, su)
            pltpu.async_copy(v_hbm.at[dst], vr, sv)

        def wait(c, ur, vr, su, sv):
            src = idxall.at[0, pl.ds(c * K, K)]
            dst = idxall.at[1, pl.ds(c * K, K)]
            pltpu.make_async_copy(u_hbm.at[src], ur, su).wait()
            pltpu.make_async_copy(v_hbm.at[dst], vr, sv).wait()

        def compute(c, ur, vr):
            def group_body(g, carry):
                out_vec = jnp.zeros((L,), jnp.float32)
                for t in range(L):
                    i = g * L + t
                    acc = jnp.zeros((L,), jnp.float32)
                    for j in range(NBB):
                        ubf = plsc.bitcast(ur[i, pl.ds(L * j, L)], jnp.bfloat16)
                        vbf = plsc.bitcast(vr[i, pl.ds(L * j, L)], jnp.bfloat16)
                        r_ab = jnp.maximum(ubf + vbf, jnp.bfloat16(0))
                        ra, rb = plsc.unpack(
                            r_ab,
                            format=plsc.PackFormat.INTERLEAVED,
                            preferred_element_type=jnp.float32,
                        )
                        acc = acc + ra * w2a[j] + rb * w2b[j]
                    out_vec = jnp.where(lane == t, _lane_sum(acc, lane), out_vec)
                outall[pl.ds(c * K + g * L, L)] = out_vec + b2vec
                return carry

            lax.fori_loop(0, K // L, group_body, 0, unroll=False)

        bufs = ((ua, va, su0, sv0), (ub, vb, su1, sv1))

        @pl.when(cid == 0)
        def _():
            issue(0, *bufs[0])

        def pair_body(m, carry):
            for bsel in range(2):
                c = 2 * m + bsel
                ur, vr, su, sv = bufs[bsel]
                wait(c, ur, vr, su, sv)

                @pl.when(c + 1 < CPW)
                def _():
                    issue(c + 1, *bufs[1 - bsel])

                compute(c, ur, vr)
            return carry

        lax.fori_loop(0, jnp.where(cid == 0, CPW // 2, 0), pair_body, 0,
                      unroll=False)
        nfull = n_out // EPW
        tail = n_out - nfull * EPW

        @pl.when((cid == 0) & (wid < nfull))
        def _():
            pltpu.sync_copy(outall, out_hbm.at[pl.ds(wid * EPW, EPW)])

        if tail:
            @pl.when((cid == 0) & (wid == nfull))
            def _():
                pltpu.sync_copy(
                    outall.at[pl.ds(0, tail)],
                    out_hbm.at[pl.ds(nfull * EPW, tail)],
                )

    return sc_kernel


@jax.jit
def kernel(h, edge_index, W1, b1, W2, b2):
    waT = W1[:, :H].T
    wbT = W1[:, H:].T
    u, v = _node_tables(h, waT, wbT, b1.reshape(1, H))
    e = edge_index.shape[1]
    half = 16 * EPW
    e_pad = 2 * half
    ei = edge_index.astype(jnp.int32)
    ei = jnp.pad(ei, ((0, 0), (0, e_pad - e)))
    w2 = _pack_halves(W2.reshape(1, H)).reshape(HP)
    b2v = jnp.broadcast_to(b2.reshape(1), (L,)).astype(jnp.float32)
    s1 = _sc_edge_kernel(0, half)(u, v, ei, w2, b2v)
    s2 = _sc_edge_kernel(half, e - half)(u, v, ei, w2, b2v)
    return jnp.concatenate([s1, s2])


# speedup vs baseline: 1.3305x; 1.3305x over previous
import functools

import jax
import jax.numpy as jnp
from jax import lax
from jax.experimental import pallas as pl
from jax.experimental.pallas import tpu as pltpu
from jax.experimental.pallas import tpu_sc as plsc

H = 256
HP = H // 2
L = 16
NBB = H // 32
K = 128
CPW = 20
EPW = K * CPW

_GATHER_DNUMS = lax.GatherDimensionNumbers(
    offset_dims=(), collapsed_slice_dims=(0,), start_index_map=(0,)
)


def _lane_shuffle(x, perm):
    return lax.gather(
        x, perm[:, None], _GATHER_DNUMS, slice_sizes=(1,),
        mode=lax.GatherScatterMode.PROMISE_IN_BOUNDS,
    )


def _lane_sum(x, lane):
    for sh in (8, 4, 2, 1):
        x = x + _lane_shuffle(x, (lane + sh) & (L - 1))
    return x


def _bf16x2_to_f32(x_bf32):
    xi = plsc.bitcast(x_bf32, jnp.int32)
    even = plsc.bitcast(lax.shift_left(xi, 16), jnp.float32)
    odd = plsc.bitcast(
        lax.bitwise_and(xi, jnp.int32(-65536)), jnp.float32
    )
    return even, odd


def _pack_halves(x):
    lo = lax.bitcast_convert_type(
        x[:, :HP].astype(jnp.bfloat16), jnp.uint16
    ).astype(jnp.int32)
    hi = lax.bitcast_convert_type(
        x[:, HP:].astype(jnp.bfloat16), jnp.uint16
    ).astype(jnp.int32)
    return lo | (hi << 16)


def _tc_body(h_ref, wa_ref, wb_ref, b1_ref, u_ref, v_ref):
    hb = h_ref[...]
    u_ref[...] = _pack_halves(
        jnp.dot(hb, wa_ref[...], preferred_element_type=jnp.float32)
    )
    v_ref[...] = _pack_halves(
        jnp.dot(hb, wb_ref[...], preferred_element_type=jnp.float32)
        + b1_ref[...]
    )


def _node_tables(h, waT, wbT, b1):
    n = h.shape[0]
    blk = 1000
    grid = n // blk
    return pl.pallas_call(
        _tc_body,
        grid=(grid,),
        in_specs=[
            pl.BlockSpec((blk, H), lambda i: (i, 0)),
            pl.BlockSpec((H, H), lambda i: (0, 0)),
            pl.BlockSpec((H, H), lambda i: (0, 0)),
            pl.BlockSpec((1, H), lambda i: (0, 0)),
        ],
        out_specs=[
            pl.BlockSpec((blk, HP), lambda i: (i, 0)),
            pl.BlockSpec((blk, HP), lambda i: (i, 0)),
        ],
        out_shape=[
            jax.ShapeDtypeStruct((n, HP), jnp.int32),
            jax.ShapeDtypeStruct((n, HP), jnp.int32),
        ],
    )(h, waT, wbT, b1)


def _sc_edge_kernel(base_edge, n_out):
    info = plsc.get_sparse_core_info()
    nc, ns = info.num_cores, info.num_subcores
    nw = nc * ns

    mesh = plsc.VectorSubcoreMesh(core_axis_name="c", subcore_axis_name="s")

    @functools.partial(
        pl.kernel,
        out_type=jax.ShapeDtypeStruct((n_out,), jnp.float32),
        mesh=mesh,
        compiler_params=pltpu.CompilerParams(needs_layout_passes=False),
        scratch_types=[
            pltpu.VMEM((2, EPW), jnp.int32),
            pltpu.VMEM((K, HP), jnp.int32),
            pltpu.VMEM((K, HP), jnp.int32),
            pltpu.VMEM((K, HP), jnp.int32),
            pltpu.VMEM((K, HP), jnp.int32),
            pltpu.VMEM((HP,), jnp.int32),
            pltpu.VMEM((L,), jnp.float32),
            pltpu.VMEM((EPW,), jnp.float32),
            pltpu.SemaphoreType.DMA,
            pltpu.SemaphoreType.DMA,
            pltpu.SemaphoreType.DMA,
            pltpu.SemaphoreType.DMA,
        ],
    )
    def sc_kernel(u_hbm, v_hbm, ei_hbm, w2_hbm, b2_hbm, out_hbm,
                  idxall, ua, ub, va, vb, w2v, b2v, outall,
                  su0, su1, sv0, sv1):
        wid = lax.axis_index("s") * nc + lax.axis_index("c")
        span = base_edge + wid * EPW
        pltpu.sync_copy(w2_hbm, w2v)
        pltpu.sync_copy(b2_hbm, b2v)
        pltpu.sync_copy(ei_hbm.at[:, pl.ds(span, EPW)], idxall)

        lane = lax.iota(jnp.int32, L)
        b2vec = b2v[...]
        w2a = []
        w2b = []
        for j in range(NBB):
            w2bf = plsc.bitcast(w2v[pl.ds(L * j, L)], jnp.bfloat16)
            a, b = plsc.unpack(
                w2bf,
                format=plsc.PackFormat.INTERLEAVED,
                preferred_element_type=jnp.float32,
            )
            w2a.append(a)
            w2b.append(b)

        def issue(c, ur, vr, su, sv):
            src = idxall.at[0, pl.ds(c * K, K)]
            dst = idxall.at[1, pl.ds(c * K, K)]
            pltpu.async_copy(u_hbm.at[src], ur, su)
            pltpu.async_copy(v_hbm.at[dst], vr, sv)

        def wait(c, ur, vr, su, sv):
            src = idxall.at[0, pl.ds(c * K, K)]
            dst = idxall.at[1, pl.ds(c * K, K)]
            pltpu.make_async_copy(u_hbm.at[src], ur, su).wait()
            pltpu.make_async_copy(v_hbm.at[dst], vr, sv).wait()

        def compute(c, ur, vr):
            def group_body(g, carry):
                out_vec = jnp.zeros((L,), jnp.float32)
                for t in range(L):
                    i = g * L + t
                    acc = jnp.zeros((L,), jnp.float32)
                    for j in range(NBB):
                        ubf = plsc.bitcast(ur[i, pl.ds(L * j, L)], jnp.bfloat16)
                        vbf = plsc.bitcast(vr[i, pl.ds(L * j, L)], jnp.bfloat16)
                        r_ab = jnp.maximum(ubf + vbf, jnp.bfloat16(0))
                        ra, rb = plsc.unpack(
                            r_ab,
                            format=plsc.PackFormat.INTERLEAVED,
                            preferred_element_type=jnp.float32,
                        )
                        acc = acc + ra * w2a[j] + rb * w2b[j]
                    out_vec = jnp.where(lane == t, _lane_sum(acc, lane), out_vec)
                outall[pl.ds(c * K + g * L, L)] = out_vec + b2vec
                return carry

            lax.fori_loop(0, K // L, group_body, 0, unroll=False)

        bufs = ((ua, va, su0, sv0), (ub, vb, su1, sv1))
        issue(0, *bufs[0])

        def pair_body(m, carry):
            for bsel in range(2):
                c = 2 * m + bsel
                ur, vr, su, sv = bufs[bsel]
                wait(c, ur, vr, su, sv)

                @pl.when(c + 1 < CPW)
                def _():
                    issue(c + 1, *bufs[1 - bsel])

                compute(c, ur, vr)
            return carry

        lax.fori_loop(0, CPW // 2, pair_body, 0, unroll=False)
        nfull = n_out // EPW
        tail = n_out - nfull * EPW

        @pl.when(wid < nfull)
        def _():
            pltpu.sync_copy(outall, out_hbm.at[pl.ds(wid * EPW, EPW)])

        if tail:
            @pl.when(wid == nfull)
            def _():
                pltpu.sync_copy(
                    outall.at[pl.ds(0, tail)],
                    out_hbm.at[pl.ds(nfull * EPW, tail)],
                )

    return sc_kernel


@jax.jit
def kernel(h, edge_index, W1, b1, W2, b2):
    waT = W1[:, :H].T
    wbT = W1[:, H:].T
    u, v = _node_tables(h, waT, wbT, b1.reshape(1, H))
    e = edge_index.shape[1]
    half = 32 * EPW
    e_pad = 2 * half
    ei = edge_index.astype(jnp.int32)
    ei = jnp.pad(ei, ((0, 0), (0, e_pad - e)))
    w2 = _pack_halves(W2.reshape(1, H)).reshape(HP)
    b2v = jnp.broadcast_to(b2.reshape(1), (L,)).astype(jnp.float32)
    s1 = _sc_edge_kernel(0, half)(u, v, ei, w2, b2v)
    s2 = _sc_edge_kernel(half, e - half)(u, v, ei, w2, b2v)
    return jnp.concatenate([s1, s2])
